# Initial kernel scaffold; baseline (speedup 1.0000x reference)
import jax
import jax.numpy as jnp
from jax.experimental import pallas as pl


def kernel(x, vec, edge_index, r_ij, f_ij, d_ij, ln_w, ln_b, Wq, bq, Wk, bk, Wv, bv, Wo, bo, Wvec, Wdk, bdk, Wdv, bdv):
    raise NotImplementedError("write your pallas kernel here")


# speedup vs baseline: 15.4600x; 15.4600x over previous
import numpy as np
import jax
import jax.numpy as jnp
from jax import lax
from jax.experimental import pallas as pl
from jax.experimental.pallas import tpu as pltpu
from jax.experimental.pallas import tpu_sc as plsc

HID = 256
NH = 8
HD = 32
CUTOFF = 5.0
F32 = jnp.float32


def _silu(x):
    return x * jax.nn.sigmoid(x)


def _node_pre_body(x_ref, vecf_ref, lnw_ref, lnb_ref, wq_ref, bq_ref,
                   wk_ref, bk_ref, wv_ref, bv_ref, w1_ref, w2_ref, w3_ref,
                   qd_ref, kvv_ref, vec3_ref, vdot_ref):
    xb = x_ref[...]
    mu = jnp.mean(xb, axis=1, keepdims=True)
    xc = xb - mu
    var = jnp.mean(xc * xc, axis=1, keepdims=True)
    xn = xc * lax.rsqrt(var + 1e-5) * lnw_ref[...] + lnb_ref[...]
    qd_ref[...] = jnp.dot(xn, wq_ref[...], preferred_element_type=F32) + bq_ref[...]
    kvv_ref[:, 0:HID] = jnp.dot(xn, wk_ref[...], preferred_element_type=F32) + bk_ref[...]
    kvv_ref[:, HID:4 * HID] = jnp.dot(xn, wv_ref[...], preferred_element_type=F32) + bv_ref[...]
    vecf = vecf_ref[...]
    vdot = jnp.zeros_like(xb)
    for c in range(3):
        vc = vecf[:, c * HID:(c + 1) * HID]
        v1 = jnp.dot(vc, w1_ref[...], preferred_element_type=F32)
        v2 = jnp.dot(vc, w2_ref[...], preferred_element_type=F32)
        v3 = jnp.dot(vc, w3_ref[...], preferred_element_type=F32)
        vdot = vdot + v1 * v2
        vec3_ref[:, c * HID:(c + 1) * HID] = v3
        kvv_ref[:, 4 * HID + c * HID:4 * HID + (c + 1) * HID] = vc
    vdot_ref[...] = vdot


def _node_pre(x, vecf, ln_w, ln_b, Wq, bq, Wk, bk, Wv, bv, W1, W2, W3):
    n = x.shape[0]
    BN = 400
    assert n % BN == 0
    grid = n // BN
    row = lambda i: (i, 0)
    full = lambda i: (0, 0)
    return pl.pallas_call(
        _node_pre_body,
        grid=(grid,),
        in_specs=[
            pl.BlockSpec((BN, HID), row),
            pl.BlockSpec((BN, 3 * HID), row),
            pl.BlockSpec((1, HID), full),
            pl.BlockSpec((1, HID), full),
            pl.BlockSpec((HID, HID), full),
            pl.BlockSpec((1, HID), full),
            pl.BlockSpec((HID, HID), full),
            pl.BlockSpec((1, HID), full),
            pl.BlockSpec((HID, 3 * HID), full),
            pl.BlockSpec((1, 3 * HID), full),
            pl.BlockSpec((HID, HID), full),
            pl.BlockSpec((HID, HID), full),
            pl.BlockSpec((HID, HID), full),
        ],
        out_specs=[
            pl.BlockSpec((BN, HID), row),
            pl.BlockSpec((BN, 7 * HID), row),
            pl.BlockSpec((BN, 3 * HID), row),
            pl.BlockSpec((BN, HID), row),
        ],
        out_shape=[
            jax.ShapeDtypeStruct((n, HID), F32),
            jax.ShapeDtypeStruct((n, 7 * HID), F32),
            jax.ShapeDtypeStruct((n, 3 * HID), F32),
            jax.ShapeDtypeStruct((n, HID), F32),
        ],
    )(x, vecf, ln_w, ln_b, Wq, bq, Wk, bk, Wv, bv, W1, W2, W3)


def _sc_gather_body_factory(E, EPW, CG, NCHUNK, NC):
    def body(qd_hbm, kvv_hbm, src_hbm, dst_hbm, qi_hbm, kvvj_hbm,
             idxs, idxd, bufq, bufk, sem):
        cid = lax.axis_index("c")
        sid = lax.axis_index("s")
        wid = sid * NC + cid
        base = pl.multiple_of(wid * EPW, 8)
        pltpu.sync_copy(src_hbm.at[pl.ds(base, EPW)], idxs)
        pltpu.sync_copy(dst_hbm.at[pl.ds(base, EPW)], idxd)

        def chunk(i, carry):
            off = pl.multiple_of(i * CG, 8)
            pltpu.async_copy(qd_hbm.at[idxd.at[pl.ds(off, CG)]], bufq, sem).wait()
            pltpu.sync_copy(bufq, qi_hbm.at[pl.ds(base + off, CG)])
            pltpu.async_copy(kvv_hbm.at[idxs.at[pl.ds(off, CG)]], bufk, sem).wait()
            pltpu.sync_copy(bufk, kvvj_hbm.at[pl.ds(base + off, CG)])
            return carry

        lax.fori_loop(0, NCHUNK, chunk, 0)
    return body


def _sc_gather(qd, kvv, src, dst):
    E = src.shape[0]
    NW = 32
    NC = 2
    EPW = E // NW
    CG = 40
    assert E % NW == 0 and EPW % CG == 0
    NCHUNK = EPW // CG
    mesh = plsc.VectorSubcoreMesh(core_axis_name="c", subcore_axis_name="s")
    fn = pl.kernel(
        _sc_gather_body_factory(E, EPW, CG, NCHUNK, NC),
        out_type=[
            jax.ShapeDtypeStruct((E, HID), F32),
            jax.ShapeDtypeStruct((E, 7 * HID), F32),
        ],
        mesh=mesh,
        scratch_types=[
            pltpu.VMEM((EPW,), jnp.int32),
            pltpu.VMEM((EPW,), jnp.int32),
            pltpu.VMEM((CG, HID), F32),
            pltpu.VMEM((CG, 7 * HID), F32),
            pltpu.SemaphoreType.DMA,
        ],
    )
    return fn(qd, kvv, src, dst)


def _edge_body(f_ref, r_ref, dij_ref, qi_ref, kvv_ref,
               wdk_ref, bdk_ref, wdv_ref, bdv_ref, msg_ref):
    f = f_ref[...]
    dk = _silu(jnp.dot(f, wdk_ref[...], preferred_element_type=F32) + bdk_ref[...])
    dv = _silu(jnp.dot(f, wdv_ref[...], preferred_element_type=F32) + bdv_ref[...])
    kvv = kvv_ref[...]
    kj = kvv[:, 0:HID]
    vj = kvv[:, HID:4 * HID]
    vecj = kvv[:, 4 * HID:7 * HID]
    t = qi_ref[...] * kj * dk
    rowi = lax.broadcasted_iota(jnp.int32, (HID, HID), 0) // HD
    coli = lax.broadcasted_iota(jnp.int32, (HID, HID), 1) // HD
    P = (rowi == coli).astype(F32)
    a = jnp.dot(t, P, preferred_element_type=F32)
    r = r_ref[...]
    cut = 0.5 * (jnp.cos(r * (np.pi / CUTOFF)) + 1.0) * (r < CUTOFF).astype(F32)
    attn = _silu(a) * cut
    msg_ref[:, 0:HID] = vj[:, 0:HID] * dv[:, 0:HID] * attn
    m1 = vj[:, HID:2 * HID] * dv[:, HID:2 * HID]
    m2 = vj[:, 2 * HID:3 * HID] * dv[:, 2 * HID:3 * HID]
    dij = dij_ref[...]
    for c in range(3):
        msg_ref[:, (c + 1) * HID:(c + 2) * HID] = (
            vecj[:, c * HID:(c + 1) * HID] * m1 + dij[:, c:c + 1] * m2)


def _edge_msgs(f_ij, r2, d_ij, qi, kvvj, Wdk, bdk, Wdv, bdv):
    E, NRBF = f_ij.shape
    BE = 640
    assert E % BE == 0
    grid = E // BE
    row = lambda i: (i, 0)
    full = lambda i: (0, 0)
    return pl.pallas_call(
        _edge_body,
        grid=(grid,),
        in_specs=[
            pl.BlockSpec((BE, NRBF), row),
            pl.BlockSpec((BE, 1), row),
            pl.BlockSpec((BE, 3), row),
            pl.BlockSpec((BE, HID), row),
            pl.BlockSpec((BE, 7 * HID), row),
            pl.BlockSpec((NRBF, HID), full),
            pl.BlockSpec((1, HID), full),
            pl.BlockSpec((NRBF, 3 * HID), full),
            pl.BlockSpec((1, 3 * HID), full),
        ],
        out_specs=[pl.BlockSpec((BE, 4 * HID), row)],
        out_shape=[jax.ShapeDtypeStruct((E, 4 * HID), F32)],
    )(f_ij, r2, d_ij, qi, kvvj, Wdk, bdk, Wdv, bdv)[0]


def _sc_scatter_body_factory(NP, E, EPT, CS, NCH, ZROWS, NPASS):
    def body(msg_hbm, dst3_hbm, agg_hbm, idx2, mbuf, zbuf, acc):
        cid = lax.axis_index("c")
        sid = lax.axis_index("s")
        r0 = pl.multiple_of(sid * ZROWS, 8)

        def zbody(i, carry):
            zbuf[i // 8, pl.ds((i % 8) * 16, 16)] = jnp.zeros((16,), F32)
            return carry
        lax.fori_loop(0, CS * 8, zbody, 0)

        pltpu.sync_copy(dst3_hbm.at[sid], idx2)
        for p in range(NPASS):
            ch0 = pl.multiple_of((cid * NPASS + p) * 128, 128)

            def zcopy(j, carry):
                pltpu.sync_copy(zbuf, acc.at[pl.ds(r0 + j * CS, CS)])
                return carry
            lax.fori_loop(0, ZROWS // CS, zcopy, 0)
            plsc.subcore_barrier()

            def sbody(i, carry):
                e0 = pl.multiple_of(sid * EPT + i * CS, 8)
                pltpu.sync_copy(msg_hbm.at[pl.ds(e0, CS), pl.ds(ch0, 128)], mbuf)
                pltpu.sync_copy(mbuf, acc.at[idx2.at[i]], add=True)
                return carry
            lax.fori_loop(0, NCH, sbody, 0)
            plsc.subcore_barrier()
            pltpu.sync_copy(acc.at[pl.ds(r0, ZROWS)],
                            agg_hbm.at[pl.ds(r0, ZROWS), pl.ds(ch0, 128)])
    return body


def _sc_scatter(msg, dst3, NP):
    E = msg.shape[0]
    NT = 16
    CS = 80
    EPT = E // NT
    NCH = EPT // CS
    ZROWS = NP // NT
    NPASS = 4
    assert E % NT == 0 and EPT % CS == 0 and NP % NT == 0 and ZROWS % 8 == 0
    mesh = plsc.VectorSubcoreMesh(core_axis_name="c", subcore_axis_name="s")
    fn = pl.kernel(
        _sc_scatter_body_factory(NP, E, EPT, CS, NCH, ZROWS, NPASS),
        out_type=[jax.ShapeDtypeStruct((NP, 4 * HID), F32)],
        mesh=mesh,
        scratch_types=[
            pltpu.VMEM((NCH, CS), jnp.int32),
            pltpu.VMEM((CS, 128), F32),
            pltpu.VMEM((CS, 128), F32),
            pltpu.VMEM_SHARED((NP, 128), F32),
        ],
    )
    return fn(msg, dst3)[0]


def _node_post_body(agg_ref, vdot_ref, vec3_ref, wo_ref, bo_ref,
                    dx_ref, dvec_ref):
    agg = agg_ref[...]
    o = jnp.dot(agg[:, 0:HID], wo_ref[...], preferred_element_type=F32) + bo_ref[...]
    o1 = o[:, 0:HID]
    o2 = o[:, HID:2 * HID]
    o3 = o[:, 2 * HID:3 * HID]
    dx_ref[...] = vdot_ref[...] * o2 + o3
    vec3 = vec3_ref[...]
    for c in range(3):
        dvec_ref[:, c * HID:(c + 1) * HID] = (
            vec3[:, c * HID:(c + 1) * HID] * o1 + agg[:, (c + 1) * HID:(c + 2) * HID])


def _node_post(agg, vdot, vec3, Wo, bo):
    n = vdot.shape[0]
    BN = 400
    assert n % BN == 0
    grid = n // BN
    row = lambda i: (i, 0)
    full = lambda i: (0, 0)
    return pl.pallas_call(
        _node_post_body,
        grid=(grid,),
        in_specs=[
            pl.BlockSpec((BN, 4 * HID), row),
            pl.BlockSpec((BN, HID), row),
            pl.BlockSpec((BN, 3 * HID), row),
            pl.BlockSpec((HID, 3 * HID), full),
            pl.BlockSpec((1, 3 * HID), full),
        ],
        out_specs=[
            pl.BlockSpec((BN, HID), row),
            pl.BlockSpec((BN, 3 * HID), row),
        ],
        out_shape=[
            jax.ShapeDtypeStruct((n, HID), F32),
            jax.ShapeDtypeStruct((n, 3 * HID), F32),
        ],
    )(agg, vdot, vec3, Wo, bo)


def kernel(x, vec, edge_index, r_ij, f_ij, d_ij, ln_w, ln_b, Wq, bq, Wk, bk,
           Wv, bv, Wo, bo, Wvec, Wdk, bdk, Wdv, bdv):
    n = x.shape[0]
    E = edge_index.shape[1]
    perm = np.array([h * (3 * HD) + p * HD + dd
                     for p in range(3) for h in range(NH) for dd in range(HD)],
                    dtype=np.int32)
    Wv = Wv[:, perm]
    bv = bv[perm]
    Wdv = Wdv[:, perm]
    bdv = bdv[perm]
    vecf = vec.reshape(n, 3 * HID)
    src = edge_index[0]
    dst = edge_index[1]
    r2 = r_ij.reshape(E, 1)
    row1 = lambda a: a.reshape(1, -1)

    qd, kvv, vec3, vdot = _node_pre(
        x, vecf, row1(ln_w), row1(ln_b), Wq, row1(bq), Wk, row1(bk),
        Wv, row1(bv), Wvec[:, 0:HID], Wvec[:, HID:2 * HID], Wvec[:, 2 * HID:3 * HID])

    qi, kvvj = _sc_gather(qd, kvv, src, dst)

    msg = _edge_msgs(f_ij, r2, d_ij, qi, kvvj, Wdk, row1(bdk), Wdv, row1(bdv))

    dst3 = dst.reshape(16, E // (16 * 80), 80)
    NP = 10240
    agg = _sc_scatter(msg, dst3, NP)

    dx, dvecf = _node_post(agg, vdot, vec3, Wo, row1(bo))
    return (dx, dvecf.reshape(n, 3, HID))


# speedup vs baseline: 23.9756x; 1.5508x over previous
import numpy as np
import jax
import jax.numpy as jnp
from jax import lax
from jax.experimental import pallas as pl
from jax.experimental.pallas import tpu as pltpu
from jax.experimental.pallas import tpu_sc as plsc

HID = 256
NH = 8
HD = 32
CUTOFF = 5.0
F32 = jnp.float32


def _silu(x):
    return x * jax.nn.sigmoid(x)


def _pack_bf16_pair(a, b):
    ia = lax.bitcast_convert_type(a, jnp.int32)
    ib = lax.bitcast_convert_type(b, jnp.int32)
    ra = (ia + 0x7FFF + ((ia >> 16) & 1)) >> 16
    rb = (ib + 0x7FFF + ((ib >> 16) & 1)) & jnp.int32(-65536)
    return (ra & 0xFFFF) | rb


def _unpack_bf16_pair(x):
    lo = lax.bitcast_convert_type(x << 16, F32)
    hi = lax.bitcast_convert_type(x & jnp.int32(-65536), F32)
    return lo, hi


def _node_pre_body(x_ref, vecf_ref, lnw_ref, lnb_ref, wq_ref, bq_ref,
                   wk_ref, bk_ref, wv_ref, bv_ref, w1_ref, w2_ref, w3_ref,
                   qd_ref, kvv_ref, vec3_ref, vdot_ref):
    xb = x_ref[...]
    mu = jnp.mean(xb, axis=1, keepdims=True)
    xc = xb - mu
    var = jnp.mean(xc * xc, axis=1, keepdims=True)
    xn = xc * lax.rsqrt(var + 1e-5) * lnw_ref[...] + lnb_ref[...]
    HH = HID // 2
    q = jnp.dot(xn, wq_ref[...], preferred_element_type=F32) + bq_ref[...]
    qd_ref[...] = _pack_bf16_pair(q[:, 0:HH], q[:, HH:HID])
    k = jnp.dot(xn, wk_ref[...], preferred_element_type=F32) + bk_ref[...]
    kvv_ref[:, 0:HH] = _pack_bf16_pair(k[:, 0:HH], k[:, HH:HID])
    v = jnp.dot(xn, wv_ref[...], preferred_element_type=F32) + bv_ref[...]
    for p in range(3):
        kvv_ref[:, HH + p * HH:HH + (p + 1) * HH] = _pack_bf16_pair(
            v[:, p * HID:p * HID + HH], v[:, p * HID + HH:(p + 1) * HID])
    rowi = lax.broadcasted_iota(jnp.int32, (HID, HID), 0)
    coli = lax.broadcasted_iota(jnp.int32, (HID, HID), 1)
    s_eo = (rowi == 2 * coli - 255 * (coli >= HH)).astype(F32)
    vecf = vecf_ref[...]
    vdot = jnp.zeros_like(xb)
    for c in range(3):
        vc = vecf[:, c * HID:(c + 1) * HID]
        v1 = jnp.dot(vc, w1_ref[...], preferred_element_type=F32)
        v2 = jnp.dot(vc, w2_ref[...], preferred_element_type=F32)
        v3 = jnp.dot(vc, w3_ref[...], preferred_element_type=F32)
        vdot = vdot + v1 * v2
        vec3_ref[:, c * HID:(c + 1) * HID] = v3
        vc_eo = jnp.dot(vc, s_eo, preferred_element_type=F32)
        kvv_ref[:, 4 * HH + c * HH:4 * HH + (c + 1) * HH] = _pack_bf16_pair(
            vc_eo[:, 0:HH], vc_eo[:, HH:HID])
    vdot_ref[...] = vdot


def _node_pre(x, vecf, ln_w, ln_b, Wq, bq, Wk, bk, Wv, bv, W1, W2, W3):
    n = x.shape[0]
    BN = 400
    assert n % BN == 0
    grid = n // BN
    row = lambda i: (i, 0)
    full = lambda i: (0, 0)
    return pl.pallas_call(
        _node_pre_body,
        grid=(grid,),
        in_specs=[
            pl.BlockSpec((BN, HID), row),
            pl.BlockSpec((BN, 3 * HID), row),
            pl.BlockSpec((1, HID), full),
            pl.BlockSpec((1, HID), full),
            pl.BlockSpec((HID, HID), full),
            pl.BlockSpec((1, HID), full),
            pl.BlockSpec((HID, HID), full),
            pl.BlockSpec((1, HID), full),
            pl.BlockSpec((HID, 3 * HID), full),
            pl.BlockSpec((1, 3 * HID), full),
            pl.BlockSpec((HID, HID), full),
            pl.BlockSpec((HID, HID), full),
            pl.BlockSpec((HID, HID), full),
        ],
        out_specs=[
            pl.BlockSpec((BN, HID // 2), row),
            pl.BlockSpec((BN, 7 * HID // 2), row),
            pl.BlockSpec((BN, 3 * HID), row),
            pl.BlockSpec((BN, HID), row),
        ],
        out_shape=[
            jax.ShapeDtypeStruct((n, HID // 2), jnp.int32),
            jax.ShapeDtypeStruct((n, 7 * HID // 2), jnp.int32),
            jax.ShapeDtypeStruct((n, 3 * HID), F32),
            jax.ShapeDtypeStruct((n, HID), F32),
        ],
    )(x, vecf, ln_w, ln_b, Wq, bq, Wk, bk, Wv, bv, W1, W2, W3)


def _sc_gather_body_factory(E, EPW, CG, NCHUNK, NC):
    def body(qd_hbm, kvv_hbm, src_hbm, dst_hbm, qi_hbm, kvvj_hbm,
             idxs, idxd, bufq, bufk, sem):
        cid = lax.axis_index("c")
        sid = lax.axis_index("s")
        wid = sid * NC + cid
        base = pl.multiple_of(wid * EPW, 8)
        pltpu.sync_copy(src_hbm.at[pl.ds(base, EPW)], idxs)
        pltpu.sync_copy(dst_hbm.at[pl.ds(base, EPW)], idxd)

        def chunk(i, carry):
            off = pl.multiple_of(i * CG, 8)
            pltpu.async_copy(qd_hbm.at[idxd.at[pl.ds(off, CG)]], bufq, sem).wait()
            pltpu.sync_copy(bufq, qi_hbm.at[pl.ds(base + off, CG)])
            pltpu.async_copy(kvv_hbm.at[idxs.at[pl.ds(off, CG)]], bufk, sem).wait()
            pltpu.sync_copy(bufk, kvvj_hbm.at[pl.ds(base + off, CG)])
            return carry

        lax.fori_loop(0, NCHUNK, chunk, 0)
    return body


def _sc_gather(qd, kvv, src, dst):
    E = src.shape[0]
    NW = 32
    NC = 2
    EPW = E // NW
    CG = 40
    assert E % NW == 0 and EPW % CG == 0
    NCHUNK = EPW // CG
    mesh = plsc.VectorSubcoreMesh(core_axis_name="c", subcore_axis_name="s")
    fn = pl.kernel(
        _sc_gather_body_factory(E, EPW, CG, NCHUNK, NC),
        out_type=[
            jax.ShapeDtypeStruct((E, HID // 2), jnp.int32),
            jax.ShapeDtypeStruct((E, 7 * HID // 2), jnp.int32),
        ],
        mesh=mesh,
        scratch_types=[
            pltpu.VMEM((EPW,), jnp.int32),
            pltpu.VMEM((EPW,), jnp.int32),
            pltpu.VMEM((CG, HID // 2), jnp.int32),
            pltpu.VMEM((CG, 7 * HID // 2), jnp.int32),
            pltpu.SemaphoreType.DMA,
        ],
    )
    return fn(qd, kvv, src, dst)


def _edge_body(f_ref, r_ref, dij_ref, qi_ref, kvv_ref,
               wdk_ref, bdk_ref, wdv_ref, bdv_ref, msg_ref):
    HH = HID // 2
    f = f_ref[...]
    dk = _silu(jnp.dot(f, wdk_ref[...], preferred_element_type=F32) + bdk_ref[...])
    dv = _silu(jnp.dot(f, wdv_ref[...], preferred_element_type=F32) + bdv_ref[...])
    klo, khi = _unpack_bf16_pair(kvv_ref[...])
    qlo, qhi = _unpack_bf16_pair(qi_ref[...])
    t = (qlo * klo[:, 0:HH] * dk[:, 0:HH]
         + qhi * khi[:, 0:HH] * dk[:, HH:HID])
    rowi = lax.broadcasted_iota(jnp.int32, (HH, HH), 0) // (HD // 2)
    coli = lax.broadcasted_iota(jnp.int32, (HH, HH), 1) // (HD // 2)
    P2 = (rowi == coli).astype(F32)
    a = jnp.dot(t, P2, preferred_element_type=F32)
    r = r_ref[...]
    cut = 0.5 * (jnp.cos(r * (np.pi / CUTOFF)) + 1.0) * (r < CUTOFF).astype(F32)
    attn = _silu(a) * cut
    msg_ref[0, :, :] = klo[:, HH:2 * HH] * dv[:, 0:HH] * attn
    msg_ref[1, :, :] = khi[:, HH:2 * HH] * dv[:, HH:HID] * attn
    m1_ev = klo[:, 2 * HH:3 * HH] * dv[:, HID:HID + HH]
    m1_od = khi[:, 2 * HH:3 * HH] * dv[:, HID + HH:2 * HID]
    m2_ev = klo[:, 3 * HH:4 * HH] * dv[:, 2 * HID:2 * HID + HH]
    m2_od = khi[:, 3 * HH:4 * HH] * dv[:, 2 * HID + HH:3 * HID]
    dij = dij_ref[...]
    for c in range(3):
        dc = dij[:, c:c + 1]
        msg_ref[2 + 2 * c, :, :] = klo[:, (4 + c) * HH:(5 + c) * HH] * m1_ev + dc * m2_ev
        msg_ref[3 + 2 * c, :, :] = khi[:, (4 + c) * HH:(5 + c) * HH] * m1_od + dc * m2_od


def _edge_msgs(f_ij, r2, d_ij, qi, kvvj, Wdk, bdk, Wdv, bdv):
    E, NRBF = f_ij.shape
    BE = 640
    assert E % BE == 0
    grid = E // BE
    row = lambda i: (i, 0)
    full = lambda i: (0, 0)
    return pl.pallas_call(
        _edge_body,
        grid=(grid,),
        in_specs=[
            pl.BlockSpec((BE, NRBF), row),
            pl.BlockSpec((BE, 1), row),
            pl.BlockSpec((BE, 3), row),
            pl.BlockSpec((BE, HID // 2), row),
            pl.BlockSpec((BE, 7 * HID // 2), row),
            pl.BlockSpec((NRBF, HID), full),
            pl.BlockSpec((1, HID), full),
            pl.BlockSpec((NRBF, 3 * HID), full),
            pl.BlockSpec((1, 3 * HID), full),
        ],
        out_specs=[pl.BlockSpec((8, BE, HID // 2), lambda i: (0, i, 0))],
        out_shape=[jax.ShapeDtypeStruct((8, E, HID // 2), F32)],
    )(f_ij, r2, d_ij, qi, kvvj, Wdk, bdk, Wdv, bdv)[0]


def _sc_scatter_body_factory(NP, E, EPT, CS, NCH, ZROWS, NPASS):
    def body(msg_hbm, dst3_hbm, agg_hbm, idx2, mbuf, zbuf, acc):
        cid = lax.axis_index("c")
        sid = lax.axis_index("s")
        r0 = pl.multiple_of(sid * ZROWS, 8)

        def zbody(i, carry):
            zbuf[i // 8, pl.ds((i % 8) * 16, 16)] = jnp.zeros((16,), F32)
            return carry
        lax.fori_loop(0, CS * 8, zbody, 0)

        pltpu.sync_copy(dst3_hbm.at[sid], idx2)
        for p in range(NPASS):
            slab = cid * NPASS + p

            def zcopy(j, carry):
                pltpu.sync_copy(zbuf, acc.at[pl.ds(r0 + j * CS, CS)])
                return carry
            lax.fori_loop(0, ZROWS // CS, zcopy, 0)
            plsc.subcore_barrier()

            def sbody(i, carry):
                e0 = pl.multiple_of(sid * EPT + i * CS, 8)
                pltpu.sync_copy(msg_hbm.at[slab, pl.ds(e0, CS)], mbuf)
                pltpu.sync_copy(mbuf, acc.at[idx2.at[i]], add=True)
                return carry
            lax.fori_loop(0, NCH, sbody, 0)
            plsc.subcore_barrier()
            pltpu.sync_copy(acc.at[pl.ds(r0, ZROWS)],
                            agg_hbm.at[slab, pl.ds(r0, ZROWS)])
    return body


def _sc_scatter(msg, dst3, NP):
    E = msg.shape[1]
    NT = 16
    CS = 80
    EPT = E // NT
    NCH = EPT // CS
    ZROWS = NP // NT
    NPASS = 4
    assert E % NT == 0 and EPT % CS == 0 and NP % NT == 0 and ZROWS % 8 == 0
    mesh = plsc.VectorSubcoreMesh(core_axis_name="c", subcore_axis_name="s")
    fn = pl.kernel(
        _sc_scatter_body_factory(NP, E, EPT, CS, NCH, ZROWS, NPASS),
        out_type=[jax.ShapeDtypeStruct((8, NP, HID // 2), F32)],
        mesh=mesh,
        scratch_types=[
            pltpu.VMEM((NCH, CS), jnp.int32),
            pltpu.VMEM((CS, 128), F32),
            pltpu.VMEM((CS, 128), F32),
            pltpu.VMEM_SHARED((NP, 128), F32),
        ],
    )
    return fn(msg, dst3)[0]


def _node_post_body(agg_ref, vdot_ref, vec3_ref, wo_ref, bo_ref,
                    dx_ref, dvec_ref):
    agg = agg_ref[...]
    x_agg = jnp.concatenate([agg[0], agg[1]], axis=1)
    o = jnp.dot(x_agg, wo_ref[...], preferred_element_type=F32) + bo_ref[...]
    o1 = o[:, 0:HID]
    o2 = o[:, HID:2 * HID]
    o3 = o[:, 2 * HID:3 * HID]
    dx_ref[...] = vdot_ref[...] * o2 + o3
    HH = HID // 2
    rowi = lax.broadcasted_iota(jnp.int32, (HID, HID), 0)
    coli = lax.broadcasted_iota(jnp.int32, (HID, HID), 1)
    p_il = (coli == 2 * rowi - 255 * (rowi >= HH)).astype(F32)
    vec3 = vec3_ref[...]
    for c in range(3):
        vag = jnp.concatenate([agg[2 + 2 * c], agg[3 + 2 * c]], axis=1)
        vag = jnp.dot(vag, p_il, preferred_element_type=F32)
        dvec_ref[:, c * HID:(c + 1) * HID] = (
            vec3[:, c * HID:(c + 1) * HID] * o1 + vag)


def _node_post(agg, vdot, vec3, Wo, bo):
    n = vdot.shape[0]
    BN = 400
    assert n % BN == 0
    grid = n // BN
    row = lambda i: (i, 0)
    full = lambda i: (0, 0)
    return pl.pallas_call(
        _node_post_body,
        grid=(grid,),
        in_specs=[
            pl.BlockSpec((8, BN, HID // 2), lambda i: (0, i, 0)),
            pl.BlockSpec((BN, HID), row),
            pl.BlockSpec((BN, 3 * HID), row),
            pl.BlockSpec((HID, 3 * HID), full),
            pl.BlockSpec((1, 3 * HID), full),
        ],
        out_specs=[
            pl.BlockSpec((BN, HID), row),
            pl.BlockSpec((BN, 3 * HID), row),
        ],
        out_shape=[
            jax.ShapeDtypeStruct((n, HID), F32),
            jax.ShapeDtypeStruct((n, 3 * HID), F32),
        ],
    )(agg, vdot, vec3, Wo, bo)


def kernel(x, vec, edge_index, r_ij, f_ij, d_ij, ln_w, ln_b, Wq, bq, Wk, bk,
           Wv, bv, Wo, bo, Wvec, Wdk, bdk, Wdv, bdv):
    n = x.shape[0]
    E = edge_index.shape[1]
    perm = np.array([h * (3 * HD) + p * HD + dd
                     for p in range(3) for h in range(NH) for dd in range(HD)],
                    dtype=np.int32)
    eo = np.concatenate([np.arange(0, HID, 2), np.arange(1, HID, 2)]).astype(np.int32)
    perm2 = perm.reshape(3, HID)[:, eo].reshape(3 * HID)
    Wv = Wv[:, perm2]
    bv = bv[perm2]
    Wdv = Wdv[:, perm2]
    bdv = bdv[perm2]
    Wq = Wq[:, eo]
    bq = bq[eo]
    Wk = Wk[:, eo]
    bk = bk[eo]
    Wdk = Wdk[:, eo]
    bdk = bdk[eo]
    Wo = Wo[eo, :]
    vecf = vec.reshape(n, 3 * HID)
    src = edge_index[0]
    dst = edge_index[1]
    r2 = r_ij.reshape(E, 1)
    row1 = lambda a: a.reshape(1, -1)

    qd, kvv, vec3, vdot = _node_pre(
        x, vecf, row1(ln_w), row1(ln_b), Wq, row1(bq), Wk, row1(bk),
        Wv, row1(bv), Wvec[:, 0:HID], Wvec[:, HID:2 * HID], Wvec[:, 2 * HID:3 * HID])

    qi, kvvj = _sc_gather(qd, kvv, src, dst)

    msg = _edge_msgs(f_ij, r2, d_ij, qi, kvvj, Wdk, row1(bdk), Wdv, row1(bdv))

    dst3 = dst.reshape(16, E // (16 * 80), 80)
    NP = 10240
    agg = _sc_scatter(msg, dst3, NP)

    dx, dvecf = _node_post(agg, vdot, vec3, Wo, row1(bo))
    return (dx, dvecf.reshape(n, 3, HID))


# speedup vs baseline: 23.9960x; 1.0009x over previous
import numpy as np
import jax
import jax.numpy as jnp
from jax import lax
from jax.experimental import pallas as pl
from jax.experimental.pallas import tpu as pltpu
from jax.experimental.pallas import tpu_sc as plsc

HID = 256
NH = 8
HD = 32
CUTOFF = 5.0
F32 = jnp.float32


def _silu(x):
    return x * jax.nn.sigmoid(x)


def _pack_bf16_pair(a, b):
    ia = lax.bitcast_convert_type(a, jnp.int32)
    ib = lax.bitcast_convert_type(b, jnp.int32)
    ra = (ia + 0x7FFF + ((ia >> 16) & 1)) >> 16
    rb = (ib + 0x7FFF + ((ib >> 16) & 1)) & jnp.int32(-65536)
    return (ra & 0xFFFF) | rb


def _unpack_bf16_pair(x):
    lo = lax.bitcast_convert_type(x << 16, F32)
    hi = lax.bitcast_convert_type(x & jnp.int32(-65536), F32)
    return lo, hi


def _node_pre_body(x_ref, vecf_ref, lnw_ref, lnb_ref, wq_ref, bq_ref,
                   wk_ref, bk_ref, wv_ref, bv_ref, w1_ref, w2_ref, w3_ref,
                   qd_ref, kvv_ref, vec3_ref, vdot_ref):
    xb = x_ref[...]
    mu = jnp.mean(xb, axis=1, keepdims=True)
    xc = xb - mu
    var = jnp.mean(xc * xc, axis=1, keepdims=True)
    xn = xc * lax.rsqrt(var + 1e-5) * lnw_ref[...] + lnb_ref[...]
    HH = HID // 2
    q = jnp.dot(xn, wq_ref[...], preferred_element_type=F32) + bq_ref[...]
    qd_ref[...] = _pack_bf16_pair(q[:, 0:HH], q[:, HH:HID])
    k = jnp.dot(xn, wk_ref[...], preferred_element_type=F32) + bk_ref[...]
    kvv_ref[:, 0:HH] = _pack_bf16_pair(k[:, 0:HH], k[:, HH:HID])
    v = jnp.dot(xn, wv_ref[...], preferred_element_type=F32) + bv_ref[...]
    for p in range(3):
        kvv_ref[:, HH + p * HH:HH + (p + 1) * HH] = _pack_bf16_pair(
            v[:, p * HID:p * HID + HH], v[:, p * HID + HH:(p + 1) * HID])
    rowi = lax.broadcasted_iota(jnp.int32, (HID, HID), 0)
    coli = lax.broadcasted_iota(jnp.int32, (HID, HID), 1)
    s_eo = (rowi == 2 * coli - 255 * (coli >= HH)).astype(F32)
    vecf = vecf_ref[...]
    vdot = jnp.zeros_like(xb)
    for c in range(3):
        vc = vecf[:, c * HID:(c + 1) * HID]
        v1 = jnp.dot(vc, w1_ref[...], preferred_element_type=F32)
        v2 = jnp.dot(vc, w2_ref[...], preferred_element_type=F32)
        v3 = jnp.dot(vc, w3_ref[...], preferred_element_type=F32)
        vdot = vdot + v1 * v2
        vec3_ref[:, c * HID:(c + 1) * HID] = v3
        vc_eo = jnp.dot(vc, s_eo, preferred_element_type=F32)
        kvv_ref[:, 4 * HH + c * HH:4 * HH + (c + 1) * HH] = _pack_bf16_pair(
            vc_eo[:, 0:HH], vc_eo[:, HH:HID])
    vdot_ref[...] = vdot


def _node_pre(x, vecf, ln_w, ln_b, Wq, bq, Wk, bk, Wv, bv, W1, W2, W3):
    n = x.shape[0]
    BN = 400
    assert n % BN == 0
    grid = n // BN
    row = lambda i: (i, 0)
    full = lambda i: (0, 0)
    return pl.pallas_call(
        _node_pre_body,
        grid=(grid,),
        in_specs=[
            pl.BlockSpec((BN, HID), row),
            pl.BlockSpec((BN, 3 * HID), row),
            pl.BlockSpec((1, HID), full),
            pl.BlockSpec((1, HID), full),
            pl.BlockSpec((HID, HID), full),
            pl.BlockSpec((1, HID), full),
            pl.BlockSpec((HID, HID), full),
            pl.BlockSpec((1, HID), full),
            pl.BlockSpec((HID, 3 * HID), full),
            pl.BlockSpec((1, 3 * HID), full),
            pl.BlockSpec((HID, HID), full),
            pl.BlockSpec((HID, HID), full),
            pl.BlockSpec((HID, HID), full),
        ],
        out_specs=[
            pl.BlockSpec((BN, HID // 2), row),
            pl.BlockSpec((BN, 7 * HID // 2), row),
            pl.BlockSpec((BN, 3 * HID), row),
            pl.BlockSpec((BN, HID), row),
        ],
        out_shape=[
            jax.ShapeDtypeStruct((n, HID // 2), jnp.int32),
            jax.ShapeDtypeStruct((n, 7 * HID // 2), jnp.int32),
            jax.ShapeDtypeStruct((n, 3 * HID), F32),
            jax.ShapeDtypeStruct((n, HID), F32),
        ],
    )(x, vecf, ln_w, ln_b, Wq, bq, Wk, bk, Wv, bv, W1, W2, W3)


def _sc_gather_body_factory(E, EPW, CG, NCHUNK, NC):
    def body(qd_hbm, kvv_hbm, src_hbm, dst_hbm, qi_hbm, kvvj_hbm,
             idxs, idxd, bufq, bufk,
             gq0, gq1, gk0, gk1, wq0, wq1, wk0, wk1):
        gq = (gq0, gq1)
        gk = (gk0, gk1)
        wq = (wq0, wq1)
        wk = (wk0, wk1)
        cid = lax.axis_index("c")
        sid = lax.axis_index("s")
        wid = sid * NC + cid
        base = pl.multiple_of(wid * EPW, 8)
        pltpu.sync_copy(src_hbm.at[pl.ds(base, EPW)], idxs)
        pltpu.sync_copy(dst_hbm.at[pl.ds(base, EPW)], idxd)

        def g_copies(c, b):
            off = pl.multiple_of(c * CG, 8)
            return (pltpu.make_async_copy(qd_hbm.at[idxd.at[pl.ds(off, CG)]], bufq.at[b], gq[b]),
                    pltpu.make_async_copy(kvv_hbm.at[idxs.at[pl.ds(off, CG)]], bufk.at[b], gk[b]))

        def w_copies(c, b):
            off = pl.multiple_of(c * CG, 8)
            return (pltpu.make_async_copy(bufq.at[b], qi_hbm.at[pl.ds(base + off, CG)], wq[b]),
                    pltpu.make_async_copy(bufk.at[b], kvvj_hbm.at[pl.ds(base + off, CG)], wk[b]))

        def pair(jj, carry):
            for b in range(2):
                s = 2 * jj + b

                @pl.when(jnp.logical_and(s >= 2, s < NCHUNK))
                def _():
                    for cp in w_copies(s - 2, b):
                        cp.wait()

                @pl.when(s < NCHUNK)
                def _():
                    for cp in g_copies(s, b):
                        cp.start()

                @pl.when(s >= 1)
                def _():
                    for cp in g_copies(s - 1, 1 - b):
                        cp.wait()
                    for cp in w_copies(s - 1, 1 - b):
                        cp.start()
            return carry

        lax.fori_loop(0, (NCHUNK + 2) // 2, pair, 0)
        for cp in w_copies(NCHUNK - 1, (NCHUNK - 1) % 2):
            cp.wait()
        for cp in w_copies(NCHUNK - 2, NCHUNK % 2):
            cp.wait()
    return body


def _sc_gather(qd, kvv, src, dst):
    E = src.shape[0]
    NW = 32
    NC = 2
    EPW = E // NW
    CG = 40
    assert E % NW == 0 and EPW % CG == 0
    NCHUNK = EPW // CG
    assert NCHUNK % 2 == 1
    mesh = plsc.VectorSubcoreMesh(core_axis_name="c", subcore_axis_name="s")
    fn = pl.kernel(
        _sc_gather_body_factory(E, EPW, CG, NCHUNK, NC),
        out_type=[
            jax.ShapeDtypeStruct((E, HID // 2), jnp.int32),
            jax.ShapeDtypeStruct((E, 7 * HID // 2), jnp.int32),
        ],
        mesh=mesh,
        scratch_types=[
            pltpu.VMEM((EPW,), jnp.int32),
            pltpu.VMEM((EPW,), jnp.int32),
            pltpu.VMEM((2, CG, HID // 2), jnp.int32),
            pltpu.VMEM((2, CG, 7 * HID // 2), jnp.int32),
        ] + [pltpu.SemaphoreType.DMA] * 8,
    )
    return fn(qd, kvv, src, dst)


def _edge_body(f_ref, r_ref, dij_ref, qi_ref, kvv_ref,
               wdk_ref, bdk_ref, wdv_ref, bdv_ref, msg_ref):
    HH = HID // 2
    f = f_ref[...]
    dk = _silu(jnp.dot(f, wdk_ref[...], preferred_element_type=F32) + bdk_ref[...])
    dv = _silu(jnp.dot(f, wdv_ref[...], preferred_element_type=F32) + bdv_ref[...])
    klo, khi = _unpack_bf16_pair(kvv_ref[...])
    qlo, qhi = _unpack_bf16_pair(qi_ref[...])
    t = (qlo * klo[:, 0:HH] * dk[:, 0:HH]
         + qhi * khi[:, 0:HH] * dk[:, HH:HID])
    rowi = lax.broadcasted_iota(jnp.int32, (HH, HH), 0) // (HD // 2)
    coli = lax.broadcasted_iota(jnp.int32, (HH, HH), 1) // (HD // 2)
    P2 = (rowi == coli).astype(F32)
    a = jnp.dot(t, P2, preferred_element_type=F32)
    r = r_ref[...]
    cut = 0.5 * (jnp.cos(r * (np.pi / CUTOFF)) + 1.0) * (r < CUTOFF).astype(F32)
    attn = _silu(a) * cut
    msg_ref[0, :, :] = klo[:, HH:2 * HH] * dv[:, 0:HH] * attn
    msg_ref[1, :, :] = khi[:, HH:2 * HH] * dv[:, HH:HID] * attn
    m1_ev = klo[:, 2 * HH:3 * HH] * dv[:, HID:HID + HH]
    m1_od = khi[:, 2 * HH:3 * HH] * dv[:, HID + HH:2 * HID]
    m2_ev = klo[:, 3 * HH:4 * HH] * dv[:, 2 * HID:2 * HID + HH]
    m2_od = khi[:, 3 * HH:4 * HH] * dv[:, 2 * HID + HH:3 * HID]
    dij = dij_ref[...]
    for c in range(3):
        dc = dij[:, c:c + 1]
        msg_ref[2 + 2 * c, :, :] = klo[:, (4 + c) * HH:(5 + c) * HH] * m1_ev + dc * m2_ev
        msg_ref[3 + 2 * c, :, :] = khi[:, (4 + c) * HH:(5 + c) * HH] * m1_od + dc * m2_od


def _edge_msgs(f_ij, r2, d_ij, qi, kvvj, Wdk, bdk, Wdv, bdv):
    E, NRBF = f_ij.shape
    BE = 640
    assert E % BE == 0
    grid = E // BE
    row = lambda i: (i, 0)
    full = lambda i: (0, 0)
    return pl.pallas_call(
        _edge_body,
        grid=(grid,),
        in_specs=[
            pl.BlockSpec((BE, NRBF), row),
            pl.BlockSpec((BE, 1), row),
            pl.BlockSpec((BE, 3), row),
            pl.BlockSpec((BE, HID // 2), row),
            pl.BlockSpec((BE, 7 * HID // 2), row),
            pl.BlockSpec((NRBF, HID), full),
            pl.BlockSpec((1, HID), full),
            pl.BlockSpec((NRBF, 3 * HID), full),
            pl.BlockSpec((1, 3 * HID), full),
        ],
        out_specs=[pl.BlockSpec((8, BE, HID // 2), lambda i: (0, i, 0))],
        out_shape=[jax.ShapeDtypeStruct((8, E, HID // 2), F32)],
    )(f_ij, r2, d_ij, qi, kvvj, Wdk, bdk, Wdv, bdv)[0]


def _sc_scatter_body_factory(NP, E, EPT, CS, NCH, ZROWS, NPASS):
    def body(msg_hbm, dst3_hbm, agg_hbm, idx2, mbuf, zbuf, acc, ls0, ls1):
        ls = (ls0, ls1)
        cid = lax.axis_index("c")
        sid = lax.axis_index("s")
        r0 = pl.multiple_of(sid * ZROWS, 8)

        def zbody(i, carry):
            zbuf[i // 8, pl.ds((i % 8) * 16, 16)] = jnp.zeros((16,), F32)
            return carry
        lax.fori_loop(0, CS * 8, zbody, 0)

        pltpu.sync_copy(dst3_hbm.at[sid], idx2)
        for p in range(NPASS):
            slab = cid * NPASS + p

            def load_cp(i, b):
                e0 = pl.multiple_of(sid * EPT + i * CS, 8)
                return pltpu.make_async_copy(
                    msg_hbm.at[slab, pl.ds(e0, CS)], mbuf.at[b], ls[b])

            def zcopy(j, carry):
                pltpu.sync_copy(zbuf, acc.at[pl.ds(r0 + j * CS, CS)])
                return carry
            lax.fori_loop(0, ZROWS // CS, zcopy, 0)
            plsc.subcore_barrier()

            load_cp(0, 0).start()

            def pair(jj, carry):
                for b in range(2):
                    s = 2 * jj + b

                    @pl.when(s < NCH - 1)
                    def _():
                        load_cp(s + 1, 1 - b).start()

                    @pl.when(s < NCH)
                    def _():
                        load_cp(s, b).wait()
                        pltpu.sync_copy(mbuf.at[b], acc.at[idx2.at[s]], add=True)
                return carry
            lax.fori_loop(0, (NCH + 1) // 2, pair, 0)
            plsc.subcore_barrier()
            pltpu.sync_copy(acc.at[pl.ds(r0, ZROWS)],
                            agg_hbm.at[slab, pl.ds(r0, ZROWS)])
    return body


def _sc_scatter(msg, dst3, NP):
    E = msg.shape[1]
    NT = 16
    CS = 80
    EPT = E // NT
    NCH = EPT // CS
    ZROWS = NP // NT
    NPASS = 4
    assert E % NT == 0 and EPT % CS == 0 and NP % NT == 0 and ZROWS % 8 == 0
    mesh = plsc.VectorSubcoreMesh(core_axis_name="c", subcore_axis_name="s")
    fn = pl.kernel(
        _sc_scatter_body_factory(NP, E, EPT, CS, NCH, ZROWS, NPASS),
        out_type=[jax.ShapeDtypeStruct((8, NP, HID // 2), F32)],
        mesh=mesh,
        scratch_types=[
            pltpu.VMEM((NCH, CS), jnp.int32),
            pltpu.VMEM((2, CS, 128), F32),
            pltpu.VMEM((CS, 128), F32),
            pltpu.VMEM_SHARED((NP, 128), F32),
            pltpu.SemaphoreType.DMA,
            pltpu.SemaphoreType.DMA,
        ],
    )
    return fn(msg, dst3)[0]


def _node_post_body(agg_ref, vdot_ref, vec3_ref, wo_ref, bo_ref,
                    dx_ref, dvec_ref):
    agg = agg_ref[...]
    x_agg = jnp.concatenate([agg[0], agg[1]], axis=1)
    o = jnp.dot(x_agg, wo_ref[...], preferred_element_type=F32) + bo_ref[...]
    o1 = o[:, 0:HID]
    o2 = o[:, HID:2 * HID]
    o3 = o[:, 2 * HID:3 * HID]
    dx_ref[...] = vdot_ref[...] * o2 + o3
    HH = HID // 2
    rowi = lax.broadcasted_iota(jnp.int32, (HID, HID), 0)
    coli = lax.broadcasted_iota(jnp.int32, (HID, HID), 1)
    p_il = (coli == 2 * rowi - 255 * (rowi >= HH)).astype(F32)
    vec3 = vec3_ref[...]
    for c in range(3):
        vag = jnp.concatenate([agg[2 + 2 * c], agg[3 + 2 * c]], axis=1)
        vag = jnp.dot(vag, p_il, preferred_element_type=F32)
        dvec_ref[:, c * HID:(c + 1) * HID] = (
            vec3[:, c * HID:(c + 1) * HID] * o1 + vag)


def _node_post(agg, vdot, vec3, Wo, bo):
    n = vdot.shape[0]
    BN = 400
    assert n % BN == 0
    grid = n // BN
    row = lambda i: (i, 0)
    full = lambda i: (0, 0)
    return pl.pallas_call(
        _node_post_body,
        grid=(grid,),
        in_specs=[
            pl.BlockSpec((8, BN, HID // 2), lambda i: (0, i, 0)),
            pl.BlockSpec((BN, HID), row),
            pl.BlockSpec((BN, 3 * HID), row),
            pl.BlockSpec((HID, 3 * HID), full),
            pl.BlockSpec((1, 3 * HID), full),
        ],
        out_specs=[
            pl.BlockSpec((BN, HID), row),
            pl.BlockSpec((BN, 3 * HID), row),
        ],
        out_shape=[
            jax.ShapeDtypeStruct((n, HID), F32),
            jax.ShapeDtypeStruct((n, 3 * HID), F32),
        ],
    )(agg, vdot, vec3, Wo, bo)


def kernel(x, vec, edge_index, r_ij, f_ij, d_ij, ln_w, ln_b, Wq, bq, Wk, bk,
           Wv, bv, Wo, bo, Wvec, Wdk, bdk, Wdv, bdv):
    n = x.shape[0]
    E = edge_index.shape[1]
    perm = np.array([h * (3 * HD) + p * HD + dd
                     for p in range(3) for h in range(NH) for dd in range(HD)],
                    dtype=np.int32)
    eo = np.concatenate([np.arange(0, HID, 2), np.arange(1, HID, 2)]).astype(np.int32)
    perm2 = perm.reshape(3, HID)[:, eo].reshape(3 * HID)
    Wv = Wv[:, perm2]
    bv = bv[perm2]
    Wdv = Wdv[:, perm2]
    bdv = bdv[perm2]
    Wq = Wq[:, eo]
    bq = bq[eo]
    Wk = Wk[:, eo]
    bk = bk[eo]
    Wdk = Wdk[:, eo]
    bdk = bdk[eo]
    Wo = Wo[eo, :]
    vecf = vec.reshape(n, 3 * HID)
    src = edge_index[0]
    dst = edge_index[1]
    r2 = r_ij.reshape(E, 1)
    row1 = lambda a: a.reshape(1, -1)

    qd, kvv, vec3, vdot = _node_pre(
        x, vecf, row1(ln_w), row1(ln_b), Wq, row1(bq), Wk, row1(bk),
        Wv, row1(bv), Wvec[:, 0:HID], Wvec[:, HID:2 * HID], Wvec[:, 2 * HID:3 * HID])

    qi, kvvj = _sc_gather(qd, kvv, src, dst)

    msg = _edge_msgs(f_ij, r2, d_ij, qi, kvvj, Wdk, row1(bdk), Wdv, row1(bdv))

    dst3 = dst.reshape(16, E // (16 * 80), 80)
    NP = 10240
    agg = _sc_scatter(msg, dst3, NP)

    dx, dvecf = _node_post(agg, vdot, vec3, Wo, row1(bo))
    return (dx, dvecf.reshape(n, 3, HID))
